# Initial kernel scaffold; baseline (speedup 1.0000x reference)
import jax
import jax.numpy as jnp
from jax.experimental import pallas as pl


def kernel(feat, node_id, edge_index, emb_table, W1, b1, W2, b2, sc1_w, sc1_b, sc2_w, sc2_b, sc3_w, sc3_b, bn1_g, bn1_b, bn2_g, bn2_b, bn3_g, bn3_b, Wih_f, Whh_f, bih_f, bhh_f, Wih_r, Whh_r, bih_r, bhh_r, act1_w, act1_b, act2_w, act2_b, con1_w, con1_b, con2_w, con2_b):
    raise NotImplementedError("write your pallas kernel here")


# speedup vs baseline: 2.8388x; 2.8388x over previous
import functools
import jax
import jax.numpy as jnp
from jax import lax
from jax.experimental import pallas as pl
from jax.experimental.pallas import tpu as pltpu
from jax.experimental.pallas import tpu_sc as plsc

N = 50000
E = 1600000
NC = 2
NS = 16
L = 16

ECHUNK = 128
NCHUNKS = 12544
E_PAD = NCHUNKS * ECHUNK
TRASH_NODE = N

NROWS_PAD = 50048
HALF = 25000
ACC_COPY = 25088
ACC_ROWS = 25096
CPY = 112

NID_CHUNKS = 416
NID_PAD = NID_CHUNKS * ECHUNK

@functools.cache
def _get_mesh():
    return plsc.VectorSubcoreMesh(core_axis_name="c", subcore_axis_name="s",
                                  num_cores=NC, num_subcores=NS)


def _emb_body(nid_hbm, emb_hbm, emb_out_hbm, nid_v, erow_v, sem):
    cid = lax.axis_index("c")
    sid = lax.axis_index("s")
    wid = sid * NC + cid

    def emb_step(c, _):
        g = wid * (NID_CHUNKS // (NC * NS)) + c
        pltpu.sync_copy(nid_hbm.at[pl.ds(g, 1)], nid_v)
        pltpu.async_copy(emb_hbm.at[nid_v.at[0]], erow_v, sem).wait()
        pltpu.sync_copy(erow_v, emb_out_hbm.at[pl.ds(g * ECHUNK, ECHUNK)])
        return _

    lax.fori_loop(0, NID_CHUNKS // (NC * NS), emb_step, 0)


def _emb_call(nid2d, emb_pad):
    kfn = pl.kernel(
        _emb_body,
        out_type=jax.ShapeDtypeStruct((NID_PAD, 16), jnp.float32),
        mesh=_get_mesh(),
        compiler_params=pltpu.CompilerParams(use_tc_tiling_on_sc=False),
        scratch_types=[
            pltpu.VMEM((1, ECHUNK), jnp.int32),
            pltpu.VMEM((ECHUNK, 16), jnp.float32),
            pltpu.SemaphoreType.DMA,
        ],
    )
    return kfn(nid2d, emb_pad)


def _agg_body(D, xs_hbm, src_hbm, dst_hbm, zeros_hbm, out_hbm,
              acc, src_v, dst_v, didx_v, row0, row1, cbuf, sem0, sem1):
    cid = lax.axis_index("c")
    sid = lax.axis_index("s")
    base = cid * HALF
    sems = (sem0, sem1)
    rows = (row0, row1)

    pltpu.sync_copy(zeros_hbm, cbuf)
    nz = ACC_ROWS - ACC_COPY
    for k in range(14):
        pltpu.sync_copy(cbuf, acc.at[pl.ds(sid * 1568 + k * CPY, CPY)])

    @pl.when(sid == NS - 1)
    def _():
        pltpu.sync_copy(cbuf.at[pl.ds(0, nz)], acc.at[pl.ds(ACC_COPY, nz)])

    plsc.subcore_barrier()

    nblk = NCHUNKS // NS // 8
    b0 = sid * (NCHUNKS // NS)

    def blk_step(b, _):
        c0 = b0 + b * 8
        pltpu.sync_copy(src_hbm.at[pl.ds(c0, 8)], src_v)
        pltpu.sync_copy(dst_hbm.at[pl.ds(c0, 8)], dst_v)
        for k in range(8):
            for j in range(ECHUNK // L):
                dv = dst_v[k, pl.ds(j * L, L)]
                loc = dv - base
                ok = (loc >= 0) & (loc < HALF)
                didx_v[k, pl.ds(j * L, L)] = jnp.where(ok, loc, ACC_COPY)
        cp0 = pltpu.async_copy(xs_hbm.at[src_v.at[0]], rows[0], sems[0])
        for k in range(8):
            if k < 7:
                nxt = pltpu.async_copy(
                    xs_hbm.at[src_v.at[k + 1]], rows[(k + 1) % 2],
                    sems[(k + 1) % 2])
            cp0.wait()
            pltpu.sync_copy(rows[k % 2], acc.at[didx_v.at[k]], add=True)
            if k < 7:
                cp0 = nxt
        return _

    lax.fori_loop(0, nblk, blk_step, 0)
    plsc.subcore_barrier()

    for k in range(14):
        r = sid * 1568 + k * CPY
        pltpu.sync_copy(acc.at[pl.ds(r, CPY)], cbuf)
        pltpu.sync_copy(cbuf, out_hbm.at[cid].at[pl.ds(r, CPY)])


def _agg_call(D, xs_pad, src2d, dst2d, zeros_d):
    kfn = pl.kernel(
        functools.partial(_agg_body, D),
        out_type=jax.ShapeDtypeStruct((NC, ACC_COPY, D), jnp.float32),
        mesh=_get_mesh(),
        compiler_params=pltpu.CompilerParams(use_tc_tiling_on_sc=False),
        scratch_types=[
            pltpu.VMEM_SHARED((ACC_ROWS, D), jnp.float32),
            pltpu.VMEM((8, ECHUNK), jnp.int32),
            pltpu.VMEM((8, ECHUNK), jnp.int32),
            pltpu.VMEM((8, ECHUNK), jnp.int32),
            pltpu.VMEM((ECHUNK, D), jnp.float32),
            pltpu.VMEM((ECHUNK, D), jnp.float32),
            pltpu.VMEM((CPY, D), jnp.float32),
            pltpu.SemaphoreType.DMA,
            pltpu.SemaphoreType.DMA,
        ],
    )
    return kfn(xs_pad, src2d, dst2d, zeros_d)


RB = 1000
NBLK = N // RB


def _norm_body(do_ref, di_ref, no_ref, ni_ref):
    no_ref[...] = lax.rsqrt(1.0 + do_ref[...])
    ni_ref[...] = lax.rsqrt(1.0 + di_ref[...])


def _norms(dego_col, degi_col):
    return pl.pallas_call(
        _norm_body,
        grid=(NBLK,),
        in_specs=[pl.BlockSpec((RB, 1), lambda i: (i, 0)),
                  pl.BlockSpec((RB, 1), lambda i: (i, 0))],
        out_specs=[pl.BlockSpec((RB, 1), lambda i: (i, 0)),
                   pl.BlockSpec((RB, 1), lambda i: (i, 0))],
        out_shape=[jax.ShapeDtypeStruct((N, 1), jnp.float32),
                   jax.ShapeDtypeStruct((N, 1), jnp.float32)],
    )(dego_col, degi_col)


def _xs_body(f_ref, e_ref, ns_ref, o_ref):
    ns = ns_ref[...]
    o_ref[...] = jnp.concatenate(
        [f_ref[...] * ns, e_ref[...] * ns,
         jnp.zeros((f_ref.shape[0], 3), jnp.float32)], axis=1)


def _make_xs(feat, emb10, ns_col):
    return pl.pallas_call(
        _xs_body,
        grid=(NBLK,),
        in_specs=[pl.BlockSpec((RB, 35), lambda i: (i, 0)),
                  pl.BlockSpec((RB, 10), lambda i: (i, 0)),
                  pl.BlockSpec((RB, 1), lambda i: (i, 0))],
        out_specs=pl.BlockSpec((RB, 48), lambda i: (i, 0)),
        out_shape=jax.ShapeDtypeStruct((N, 48), jnp.float32),
    )(feat, emb10, ns_col)


def _skip1_body(f_ref, e_ref, wa_ref, wb_ref, b_ref, o_ref):
    o_ref[...] = (
        jnp.dot(f_ref[...], wa_ref[...], preferred_element_type=jnp.float32, precision=lax.Precision.HIGHEST)
        + jnp.dot(e_ref[...], wb_ref[...], preferred_element_type=jnp.float32, precision=lax.Precision.HIGHEST)
        + b_ref[...])


def _skip1(feat, emb10, wa, wb, b):
    co = wa.shape[1]
    return pl.pallas_call(
        _skip1_body,
        grid=(NBLK,),
        in_specs=[pl.BlockSpec((RB, 35), lambda i: (i, 0)),
                  pl.BlockSpec((RB, 10), lambda i: (i, 0)),
                  pl.BlockSpec(wa.shape, lambda i: (0, 0)),
                  pl.BlockSpec(wb.shape, lambda i: (0, 0)),
                  pl.BlockSpec((1, co), lambda i: (0, 0))],
        out_specs=pl.BlockSpec((RB, co), lambda i: (i, 0)),
        out_shape=jax.ShapeDtypeStruct((N, co), jnp.float32),
    )(feat, emb10, wa, wb, b)


def _l1a_body(agg_ref, xs_ref, nd_ref, w_ref, b_ref, p_ref, st_ref):
    i = pl.program_id(0)
    t = (agg_ref[...] + xs_ref[...]) * nd_ref[...]
    p = jnp.dot(t, w_ref[...], preferred_element_type=jnp.float32, precision=lax.Precision.HIGHEST) + b_ref[...]
    p_ref[...] = p

    @pl.when(i == 0)
    def _():
        st_ref[...] = jnp.zeros_like(st_ref)

    st_ref[0, :] += jnp.sum(p, axis=0)
    st_ref[1, :] += jnp.sum(p * p, axis=0)


def _l1a(agg, xs, nd_col, w, b):
    ci, co = w.shape
    return pl.pallas_call(
        _l1a_body,
        grid=(NBLK,),
        in_specs=[pl.BlockSpec((RB, ci), lambda i: (i, 0)),
                  pl.BlockSpec((RB, ci), lambda i: (i, 0)),
                  pl.BlockSpec((RB, 1), lambda i: (i, 0)),
                  pl.BlockSpec((ci, co), lambda i: (0, 0)),
                  pl.BlockSpec((1, co), lambda i: (0, 0))],
        out_specs=[pl.BlockSpec((RB, co), lambda i: (i, 0)),
                   pl.BlockSpec((8, co), lambda i: (0, 0))],
        out_shape=[jax.ShapeDtypeStruct((N, co), jnp.float32),
                   jax.ShapeDtypeStruct((8, co), jnp.float32)],
    )(agg, xs, nd_col, w, b)


def _bn(p, st_ref, g_ref, b_ref):
    mu = st_ref[0, :] / N
    var = st_ref[1, :] / N - mu * mu
    return (p - mu) * lax.rsqrt(var + 1e-5) * g_ref[...] + b_ref[...]


def _l1b_body(p_ref, st_ref, g_ref, bb_ref, sk_ref, ns_ref, w2_ref,
              x1_ref, y2_ref):
    x1 = jnp.maximum(_bn(p_ref[...], st_ref, g_ref, bb_ref) + sk_ref[...], 0.0)
    x1_ref[...] = x1
    y2_ref[...] = jnp.dot(x1 * ns_ref[...], w2_ref[...],
                          preferred_element_type=jnp.float32, precision=lax.Precision.HIGHEST)


def _l1b(p, st, g, b, skip, ns_col, w2):
    ci, co = w2.shape
    return pl.pallas_call(
        _l1b_body,
        grid=(NBLK,),
        in_specs=[pl.BlockSpec((RB, ci), lambda i: (i, 0)),
                  pl.BlockSpec((8, ci), lambda i: (0, 0)),
                  pl.BlockSpec((1, ci), lambda i: (0, 0)),
                  pl.BlockSpec((1, ci), lambda i: (0, 0)),
                  pl.BlockSpec((RB, ci), lambda i: (i, 0)),
                  pl.BlockSpec((RB, 1), lambda i: (i, 0)),
                  pl.BlockSpec((ci, co), lambda i: (0, 0))],
        out_specs=[pl.BlockSpec((RB, ci), lambda i: (i, 0)),
                   pl.BlockSpec((RB, co), lambda i: (i, 0))],
        out_shape=[jax.ShapeDtypeStruct((N, ci), jnp.float32),
                   jax.ShapeDtypeStruct((N, co), jnp.float32)],
    )(p, st, g, b, skip, ns_col, w2)


def _l2a_body(agg_ref, y2_ref, nd_ref, b_ref, p_ref, st_ref):
    i = pl.program_id(0)
    p = (agg_ref[...] + y2_ref[...]) * nd_ref[...] + b_ref[...]
    p_ref[...] = p

    @pl.when(i == 0)
    def _():
        st_ref[...] = jnp.zeros_like(st_ref)

    st_ref[0, :] += jnp.sum(p, axis=0)
    st_ref[1, :] += jnp.sum(p * p, axis=0)


def _l2a(agg, y2, nd_col, b):
    co = y2.shape[1]
    return pl.pallas_call(
        _l2a_body,
        grid=(NBLK,),
        in_specs=[pl.BlockSpec((RB, co), lambda i: (i, 0)),
                  pl.BlockSpec((RB, co), lambda i: (i, 0)),
                  pl.BlockSpec((RB, 1), lambda i: (i, 0)),
                  pl.BlockSpec((1, co), lambda i: (0, 0))],
        out_specs=[pl.BlockSpec((RB, co), lambda i: (i, 0)),
                   pl.BlockSpec((8, co), lambda i: (0, 0))],
        out_shape=[jax.ShapeDtypeStruct((N, co), jnp.float32),
                   jax.ShapeDtypeStruct((8, co), jnp.float32)],
    )(agg, y2, nd_col, b)


def _l2b_body(p_ref, st_ref, g_ref, bb_ref, x1_ref, skw_ref, skb_ref,
              wf_ref, bf_ref, wr_ref, br_ref, w3_ref, b3_ref,
              gf_ref, gr_ref, sk3_ref):
    sk = jnp.dot(x1_ref[...], skw_ref[...],
                 preferred_element_type=jnp.float32, precision=lax.Precision.HIGHEST) + skb_ref[...]
    x2 = jnp.maximum(_bn(p_ref[...], st_ref, g_ref, bb_ref) + sk, 0.0)
    gf_ref[...] = jnp.dot(x2, wf_ref[...],
                          preferred_element_type=jnp.float32, precision=lax.Precision.HIGHEST) + bf_ref[...]
    gr_ref[...] = jnp.dot(x2, wr_ref[...],
                          preferred_element_type=jnp.float32, precision=lax.Precision.HIGHEST) + br_ref[...]
    sk3_ref[...] = jnp.dot(x2, w3_ref[...],
                           preferred_element_type=jnp.float32, precision=lax.Precision.HIGHEST) + b3_ref[...]


def _l2b(p, st, g, b, x1, skw, skb, wf, bf, wr, br, w3, b3):
    return pl.pallas_call(
        _l2b_body,
        grid=(NBLK,),
        in_specs=[pl.BlockSpec((RB, 64), lambda i: (i, 0)),
                  pl.BlockSpec((8, 64), lambda i: (0, 0)),
                  pl.BlockSpec((1, 64), lambda i: (0, 0)),
                  pl.BlockSpec((1, 64), lambda i: (0, 0)),
                  pl.BlockSpec((RB, 128), lambda i: (i, 0)),
                  pl.BlockSpec((128, 64), lambda i: (0, 0)),
                  pl.BlockSpec((1, 64), lambda i: (0, 0)),
                  pl.BlockSpec((64, 128), lambda i: (0, 0)),
                  pl.BlockSpec((1, 128), lambda i: (0, 0)),
                  pl.BlockSpec((64, 128), lambda i: (0, 0)),
                  pl.BlockSpec((1, 128), lambda i: (0, 0)),
                  pl.BlockSpec((64, 64), lambda i: (0, 0)),
                  pl.BlockSpec((1, 64), lambda i: (0, 0))],
        out_specs=[pl.BlockSpec((RB, 128), lambda i: (i, 0)),
                   pl.BlockSpec((RB, 128), lambda i: (i, 0)),
                   pl.BlockSpec((RB, 64), lambda i: (i, 0))],
        out_shape=[jax.ShapeDtypeStruct((N, 128), jnp.float32),
                   jax.ShapeDtypeStruct((N, 128), jnp.float32),
                   jax.ShapeDtypeStruct((N, 64), jnp.float32)],
    )(p, st, g, b, x1, skw, skb, wf, bf, wr, br, w3, b3)


def _lstm_body(gf_ref, gr_ref, wf_ref, wr_ref, of_ref, or_ref, carry_ref):
    i = pl.program_id(0)

    @pl.when(i == 0)
    def _():
        carry_ref[...] = jnp.zeros_like(carry_ref)

    hf = carry_ref[0:1, :]
    cf = carry_ref[1:2, :]
    hr = carry_ref[2:3, :]
    cr = carry_ref[3:4, :]
    wf = wf_ref[...]
    wr = wr_ref[...]

    def step(j, carry):
        hf, cf, hr, cr = carry
        zf = gf_ref[pl.ds(j, 1), :] + jnp.dot(
            hf, wf, preferred_element_type=jnp.float32, precision=lax.Precision.HIGHEST)
        zr = gr_ref[pl.ds(RB - 1 - j, 1), :] + jnp.dot(
            hr, wr, preferred_element_type=jnp.float32, precision=lax.Precision.HIGHEST)
        i_f = jax.nn.sigmoid(zf[:, 0:32])
        f_f = jax.nn.sigmoid(zf[:, 32:64])
        g_f = jnp.tanh(zf[:, 64:96])
        o_f = jax.nn.sigmoid(zf[:, 96:128])
        cf = f_f * cf + i_f * g_f
        hf = o_f * jnp.tanh(cf)
        i_r = jax.nn.sigmoid(zr[:, 0:32])
        f_r = jax.nn.sigmoid(zr[:, 32:64])
        g_r = jnp.tanh(zr[:, 64:96])
        o_r = jax.nn.sigmoid(zr[:, 96:128])
        cr = f_r * cr + i_r * g_r
        hr = o_r * jnp.tanh(cr)
        of_ref[pl.ds(j, 1), :] = hf
        or_ref[pl.ds(RB - 1 - j, 1), :] = hr
        return (hf, cf, hr, cr)

    hf, cf, hr, cr = lax.fori_loop(0, RB, step, (hf, cf, hr, cr))
    carry_ref[0:1, :] = hf
    carry_ref[1:2, :] = cf
    carry_ref[2:3, :] = hr
    carry_ref[3:4, :] = cr


def _lstm(gf, gr, wf, wr):
    return pl.pallas_call(
        _lstm_body,
        grid=(NBLK,),
        in_specs=[pl.BlockSpec((RB, 128), lambda i: (i, 0)),
                  pl.BlockSpec((RB, 128), lambda i: (NBLK - 1 - i, 0)),
                  pl.BlockSpec((32, 128), lambda i: (0, 0)),
                  pl.BlockSpec((32, 128), lambda i: (0, 0))],
        out_specs=[pl.BlockSpec((RB, 32), lambda i: (i, 0)),
                   pl.BlockSpec((RB, 32), lambda i: (NBLK - 1 - i, 0))],
        out_shape=[jax.ShapeDtypeStruct((N, 32), jnp.float32),
                   jax.ShapeDtypeStruct((N, 32), jnp.float32)],
        scratch_shapes=[pltpu.VMEM((4, 32), jnp.float32)],
    )(gf, gr, wf, wr)


def _stat64_body(hf_ref, hr_ref, st_ref):
    i = pl.program_id(0)
    h = jnp.concatenate([hf_ref[...], hr_ref[...]], axis=1)

    @pl.when(i == 0)
    def _():
        st_ref[...] = jnp.zeros_like(st_ref)

    st_ref[0, :] += jnp.sum(h, axis=0)
    st_ref[1, :] += jnp.sum(h * h, axis=0)


def _stat64(hf, hr):
    return pl.pallas_call(
        _stat64_body,
        grid=(NBLK,),
        in_specs=[pl.BlockSpec((RB, 32), lambda i: (i, 0)),
                  pl.BlockSpec((RB, 32), lambda i: (i, 0))],
        out_specs=pl.BlockSpec((8, 64), lambda i: (0, 0)),
        out_shape=jax.ShapeDtypeStruct((8, 64), jnp.float32),
    )(hf, hr)


def _head_body(hf_ref, hr_ref, st_ref, g_ref, bb_ref, sk3_ref,
               aw1_ref, ab1_ref, aw2_ref, ab2_ref,
               cw1_ref, cb1_ref, cw2_ref, cb2_ref, act_ref, con_ref):
    h = jnp.concatenate([hf_ref[...], hr_ref[...]], axis=1)
    x3 = jnp.maximum(_bn(h, st_ref, g_ref, bb_ref) + sk3_ref[...], 0.0)
    a = jnp.maximum(jnp.dot(x3, aw1_ref[...],
                            preferred_element_type=jnp.float32, precision=lax.Precision.HIGHEST)
                    + ab1_ref[...], 0.0)
    act_ref[...] = jnp.dot(a, aw2_ref[...],
                           preferred_element_type=jnp.float32, precision=lax.Precision.HIGHEST) + ab2_ref[...]
    c = jnp.maximum(jnp.dot(x3, cw1_ref[...],
                            preferred_element_type=jnp.float32, precision=lax.Precision.HIGHEST)
                    + cb1_ref[...], 0.0)
    con_ref[...] = jnp.dot(c, cw2_ref[...],
                           preferred_element_type=jnp.float32, precision=lax.Precision.HIGHEST) + cb2_ref[...]


def _heads(hf, hr, st, g, b, sk3, aw1, ab1, aw2, ab2, cw1, cb1, cw2, cb2):
    return pl.pallas_call(
        _head_body,
        grid=(NBLK,),
        in_specs=[pl.BlockSpec((RB, 32), lambda i: (i, 0)),
                  pl.BlockSpec((RB, 32), lambda i: (i, 0)),
                  pl.BlockSpec((8, 64), lambda i: (0, 0)),
                  pl.BlockSpec((1, 64), lambda i: (0, 0)),
                  pl.BlockSpec((1, 64), lambda i: (0, 0)),
                  pl.BlockSpec((RB, 64), lambda i: (i, 0)),
                  pl.BlockSpec((64, 64), lambda i: (0, 0)),
                  pl.BlockSpec((1, 64), lambda i: (0, 0)),
                  pl.BlockSpec((64, 1), lambda i: (0, 0)),
                  pl.BlockSpec((1, 1), lambda i: (0, 0)),
                  pl.BlockSpec((64, 64), lambda i: (0, 0)),
                  pl.BlockSpec((1, 64), lambda i: (0, 0)),
                  pl.BlockSpec((64, 1), lambda i: (0, 0)),
                  pl.BlockSpec((1, 1), lambda i: (0, 0))],
        out_specs=[pl.BlockSpec((RB, 1), lambda i: (i, 0)),
                   pl.BlockSpec((RB, 1), lambda i: (i, 0))],
        out_shape=[jax.ShapeDtypeStruct((N, 1), jnp.float32),
                   jax.ShapeDtypeStruct((N, 1), jnp.float32)],
    )(hf, hr, st, g, b, sk3, aw1, ab1, aw2, ab2, cw1, cb1, cw2, cb2)


def kernel(feat, node_id, edge_index, emb_table, W1, b1, W2, b2, sc1_w, sc1_b,
           sc2_w, sc2_b, sc3_w, sc3_b, bn1_g, bn1_b, bn2_g, bn2_b, bn3_g,
           bn3_b, Wih_f, Whh_f, bih_f, bhh_f, Wih_r, Whh_r, bih_r, bhh_r,
           act1_w, act1_b, act2_w, act2_b, con1_w, con1_b, con2_w, con2_b):
    f32 = jnp.float32
    src = edge_index[0].astype(jnp.int32)
    dst = edge_index[1].astype(jnp.int32)
    pad = jnp.full((E_PAD - E,), TRASH_NODE, jnp.int32)
    src2d = jnp.concatenate([src, pad]).reshape(NCHUNKS, ECHUNK)
    dst2d = jnp.concatenate([dst, pad]).reshape(NCHUNKS, ECHUNK)
    nid2d = jnp.pad(node_id.astype(jnp.int32),
                    (0, NID_PAD - N)).reshape(NID_CHUNKS, ECHUNK)
    emb_pad = jnp.pad(emb_table.astype(f32), ((0, 0), (0, 6)))

    emb_rows = _emb_call(nid2d, emb_pad)

    ones_pad = jnp.pad(jnp.ones((NROWS_PAD, 1), f32), ((0, 0), (0, 15)))
    zeros16 = jnp.zeros((CPY, 16), f32)
    degi_2 = _agg_call(16, ones_pad, src2d, dst2d, zeros16)
    dego_2 = _agg_call(16, ones_pad, dst2d, src2d, zeros16)
    degi_col = jnp.concatenate([degi_2[0, :HALF, :1], degi_2[1, :HALF, :1]])
    dego_col = jnp.concatenate([dego_2[0, :HALF, :1], dego_2[1, :HALF, :1]])
    ns_col, nd_col = _norms(dego_col, degi_col)

    emb10 = emb_rows[:N, :10]
    xs = _make_xs(feat, emb10, ns_col)
    xs_pad = jnp.pad(xs, ((0, NROWS_PAD - N), (0, 0)))

    sc1_wa = sc1_w[:35]
    sc1_wb = sc1_w[35:45]
    skip1 = _skip1(feat, emb10, sc1_wa, sc1_wb, sc1_b.reshape(1, -1))

    zeros48 = jnp.zeros((CPY, 48), f32)
    agg1_2 = _agg_call(48, xs_pad, src2d, dst2d, zeros48)
    agg1 = jnp.concatenate([agg1_2[0, :HALF], agg1_2[1, :HALF]], axis=0)

    W1p = jnp.pad(W1, ((0, 3), (0, 0)))
    p1, st1 = _l1a(agg1, xs, nd_col, W1p, b1.reshape(1, -1))
    x1, y2 = _l1b(p1, st1, bn1_g.reshape(1, -1), bn1_b.reshape(1, -1),
                  skip1, ns_col, W2)

    y2_pad = jnp.pad(y2, ((0, NROWS_PAD - N), (0, 0)))
    zeros64 = jnp.zeros((CPY, 64), f32)
    agg2_2 = _agg_call(64, y2_pad, src2d, dst2d, zeros64)
    agg2 = jnp.concatenate([agg2_2[0, :HALF], agg2_2[1, :HALF]], axis=0)

    p2, st2 = _l2a(agg2, y2, nd_col, b2.reshape(1, -1))
    gf, gr, sk3 = _l2b(
        p2, st2, bn2_g.reshape(1, -1), bn2_b.reshape(1, -1), x1,
        sc2_w, sc2_b.reshape(1, -1),
        Wih_f, (bih_f + bhh_f).reshape(1, -1),
        Wih_r, (bih_r + bhh_r).reshape(1, -1),
        sc3_w, sc3_b.reshape(1, -1))

    hs_f, hs_r = _lstm(gf, gr, Whh_f, Whh_r)
    st3 = _stat64(hs_f, hs_r)
    active, consume = _heads(
        hs_f, hs_r, st3, bn3_g.reshape(1, -1), bn3_b.reshape(1, -1), sk3,
        act1_w, act1_b.reshape(1, -1), act2_w, act2_b.reshape(1, -1),
        con1_w, con1_b.reshape(1, -1), con2_w, con2_b.reshape(1, -1))
    return (active, consume)


# speedup vs baseline: 5.1292x; 1.8068x over previous
import functools
import jax
import jax.numpy as jnp
from jax import lax
from jax.experimental import pallas as pl
from jax.experimental.pallas import tpu as pltpu
from jax.experimental.pallas import tpu_sc as plsc

N = 50000
E = 1600000
NC = 2
NS = 16
L = 16

ECHUNK = 128
NCHUNKS = 12544
E_PAD = NCHUNKS * ECHUNK
TRASH_NODE = N

NROWS_PAD = 50048
HALF = 25000
ACC_COPY = 25088
ACC_ROWS = 25096
CPY = 112

NID_CHUNKS = 416
NID_PAD = NID_CHUNKS * ECHUNK

@functools.cache
def _get_mesh():
    return plsc.VectorSubcoreMesh(core_axis_name="c", subcore_axis_name="s",
                                  num_cores=NC, num_subcores=NS)


def _emb_body(nid_hbm, emb_hbm, emb_out_hbm, nid_v, erow_v, sem):
    cid = lax.axis_index("c")
    sid = lax.axis_index("s")
    wid = sid * NC + cid

    def emb_step(c, _):
        g = wid * (NID_CHUNKS // (NC * NS)) + c
        pltpu.sync_copy(nid_hbm.at[pl.ds(g, 1)], nid_v)
        pltpu.async_copy(emb_hbm.at[nid_v.at[0]], erow_v, sem).wait()
        pltpu.sync_copy(erow_v, emb_out_hbm.at[pl.ds(g * ECHUNK, ECHUNK)])
        return _

    lax.fori_loop(0, NID_CHUNKS // (NC * NS), emb_step, 0)


def _emb_call(nid2d, emb_pad):
    kfn = pl.kernel(
        _emb_body,
        out_type=jax.ShapeDtypeStruct((NID_PAD, 16), jnp.float32),
        mesh=_get_mesh(),
        compiler_params=pltpu.CompilerParams(use_tc_tiling_on_sc=False),
        scratch_types=[
            pltpu.VMEM((1, ECHUNK), jnp.int32),
            pltpu.VMEM((ECHUNK, 16), jnp.float32),
            pltpu.SemaphoreType.DMA,
        ],
    )
    return kfn(nid2d, emb_pad)


def _agg_body(D, xs_hbm, src_hbm, dst_hbm, zeros_hbm, out_hbm,
              acc, src_v, dst_v, didx_v, row0, row1, cbuf, sem0, sem1):
    cid = lax.axis_index("c")
    sid = lax.axis_index("s")
    base = cid * HALF
    sems = (sem0, sem1)
    rows = (row0, row1)

    pltpu.sync_copy(zeros_hbm, cbuf)
    nz = ACC_ROWS - ACC_COPY
    for k in range(14):
        pltpu.sync_copy(cbuf, acc.at[pl.ds(sid * 1568 + k * CPY, CPY)])

    @pl.when(sid == NS - 1)
    def _():
        pltpu.sync_copy(cbuf.at[pl.ds(0, nz)], acc.at[pl.ds(ACC_COPY, nz)])

    plsc.subcore_barrier()

    nblk = NCHUNKS // NS // 8
    b0 = sid * (NCHUNKS // NS)

    def blk_step(b, _):
        c0 = b0 + b * 8
        pltpu.sync_copy(src_hbm.at[pl.ds(c0, 8)], src_v)
        pltpu.sync_copy(dst_hbm.at[pl.ds(c0, 8)], dst_v)
        for k in range(8):
            for j in range(ECHUNK // L):
                dv = dst_v[k, pl.ds(j * L, L)]
                loc = dv - base
                ok = (loc >= 0) & (loc < HALF)
                didx_v[k, pl.ds(j * L, L)] = jnp.where(ok, loc, ACC_COPY)
        cp0 = pltpu.async_copy(xs_hbm.at[src_v.at[0]], rows[0], sems[0])
        for k in range(8):
            if k < 7:
                nxt = pltpu.async_copy(
                    xs_hbm.at[src_v.at[k + 1]], rows[(k + 1) % 2],
                    sems[(k + 1) % 2])
            cp0.wait()
            pltpu.sync_copy(rows[k % 2], acc.at[didx_v.at[k]], add=True)
            if k < 7:
                cp0 = nxt
        return _

    lax.fori_loop(0, nblk, blk_step, 0)
    plsc.subcore_barrier()

    for k in range(14):
        r = sid * 1568 + k * CPY
        pltpu.sync_copy(acc.at[pl.ds(r, CPY)], cbuf)
        pltpu.sync_copy(cbuf, out_hbm.at[cid].at[pl.ds(r, CPY)])


def _agg_call(D, xs_pad, src2d, dst2d, zeros_d):
    kfn = pl.kernel(
        functools.partial(_agg_body, D),
        out_type=jax.ShapeDtypeStruct((NC, ACC_COPY, D), jnp.float32),
        mesh=_get_mesh(),
        compiler_params=pltpu.CompilerParams(use_tc_tiling_on_sc=False),
        scratch_types=[
            pltpu.VMEM_SHARED((ACC_ROWS, D), jnp.float32),
            pltpu.VMEM((8, ECHUNK), jnp.int32),
            pltpu.VMEM((8, ECHUNK), jnp.int32),
            pltpu.VMEM((8, ECHUNK), jnp.int32),
            pltpu.VMEM((ECHUNK, D), jnp.float32),
            pltpu.VMEM((ECHUNK, D), jnp.float32),
            pltpu.VMEM((CPY, D), jnp.float32),
            pltpu.SemaphoreType.DMA,
            pltpu.SemaphoreType.DMA,
        ],
    )
    return kfn(xs_pad, src2d, dst2d, zeros_d)


RB = 1000
NBLK = N // RB


def _norm_body(do_ref, di_ref, no_ref, ni_ref):
    no_ref[...] = lax.rsqrt(1.0 + do_ref[...])
    ni_ref[...] = lax.rsqrt(1.0 + di_ref[...])


def _norms(dego_col, degi_col):
    return pl.pallas_call(
        _norm_body,
        grid=(NBLK,),
        in_specs=[pl.BlockSpec((RB, 1), lambda i: (i, 0)),
                  pl.BlockSpec((RB, 1), lambda i: (i, 0))],
        out_specs=[pl.BlockSpec((RB, 1), lambda i: (i, 0)),
                   pl.BlockSpec((RB, 1), lambda i: (i, 0))],
        out_shape=[jax.ShapeDtypeStruct((N, 1), jnp.float32),
                   jax.ShapeDtypeStruct((N, 1), jnp.float32)],
    )(dego_col, degi_col)


def _xs_body(f_ref, e_ref, ns_ref, o_ref):
    ns = ns_ref[...]
    o_ref[...] = jnp.concatenate(
        [f_ref[...] * ns, e_ref[...] * ns,
         jnp.zeros((f_ref.shape[0], 3), jnp.float32)], axis=1)


def _make_xs(feat, emb10, ns_col):
    return pl.pallas_call(
        _xs_body,
        grid=(NBLK,),
        in_specs=[pl.BlockSpec((RB, 35), lambda i: (i, 0)),
                  pl.BlockSpec((RB, 10), lambda i: (i, 0)),
                  pl.BlockSpec((RB, 1), lambda i: (i, 0))],
        out_specs=pl.BlockSpec((RB, 48), lambda i: (i, 0)),
        out_shape=jax.ShapeDtypeStruct((N, 48), jnp.float32),
    )(feat, emb10, ns_col)


def _skip1_body(f_ref, e_ref, wa_ref, wb_ref, b_ref, o_ref):
    o_ref[...] = (
        jnp.dot(f_ref[...], wa_ref[...], preferred_element_type=jnp.float32, precision=lax.Precision.HIGHEST)
        + jnp.dot(e_ref[...], wb_ref[...], preferred_element_type=jnp.float32, precision=lax.Precision.HIGHEST)
        + b_ref[...])


def _skip1(feat, emb10, wa, wb, b):
    co = wa.shape[1]
    return pl.pallas_call(
        _skip1_body,
        grid=(NBLK,),
        in_specs=[pl.BlockSpec((RB, 35), lambda i: (i, 0)),
                  pl.BlockSpec((RB, 10), lambda i: (i, 0)),
                  pl.BlockSpec(wa.shape, lambda i: (0, 0)),
                  pl.BlockSpec(wb.shape, lambda i: (0, 0)),
                  pl.BlockSpec((1, co), lambda i: (0, 0))],
        out_specs=pl.BlockSpec((RB, co), lambda i: (i, 0)),
        out_shape=jax.ShapeDtypeStruct((N, co), jnp.float32),
    )(feat, emb10, wa, wb, b)


def _l1a_body(agg_ref, xs_ref, nd_ref, w_ref, b_ref, p_ref, st_ref):
    i = pl.program_id(0)
    t = (agg_ref[...] + xs_ref[...]) * nd_ref[...]
    p = jnp.dot(t, w_ref[...], preferred_element_type=jnp.float32, precision=lax.Precision.HIGHEST) + b_ref[...]
    p_ref[...] = p

    @pl.when(i == 0)
    def _():
        st_ref[...] = jnp.zeros_like(st_ref)

    st_ref[0, :] += jnp.sum(p, axis=0)
    st_ref[1, :] += jnp.sum(p * p, axis=0)


def _l1a(agg, xs, nd_col, w, b):
    ci, co = w.shape
    return pl.pallas_call(
        _l1a_body,
        grid=(NBLK,),
        in_specs=[pl.BlockSpec((RB, ci), lambda i: (i, 0)),
                  pl.BlockSpec((RB, ci), lambda i: (i, 0)),
                  pl.BlockSpec((RB, 1), lambda i: (i, 0)),
                  pl.BlockSpec((ci, co), lambda i: (0, 0)),
                  pl.BlockSpec((1, co), lambda i: (0, 0))],
        out_specs=[pl.BlockSpec((RB, co), lambda i: (i, 0)),
                   pl.BlockSpec((8, co), lambda i: (0, 0))],
        out_shape=[jax.ShapeDtypeStruct((N, co), jnp.float32),
                   jax.ShapeDtypeStruct((8, co), jnp.float32)],
    )(agg, xs, nd_col, w, b)


def _bn(p, st_ref, g_ref, b_ref):
    mu = st_ref[0, :] / N
    var = st_ref[1, :] / N - mu * mu
    return (p - mu) * lax.rsqrt(var + 1e-5) * g_ref[...] + b_ref[...]


def _l1b_body(p_ref, st_ref, g_ref, bb_ref, sk_ref, ns_ref, w2_ref,
              x1_ref, y2_ref):
    x1 = jnp.maximum(_bn(p_ref[...], st_ref, g_ref, bb_ref) + sk_ref[...], 0.0)
    x1_ref[...] = x1
    y2_ref[...] = jnp.dot(x1 * ns_ref[...], w2_ref[...],
                          preferred_element_type=jnp.float32, precision=lax.Precision.HIGHEST)


def _l1b(p, st, g, b, skip, ns_col, w2):
    ci, co = w2.shape
    return pl.pallas_call(
        _l1b_body,
        grid=(NBLK,),
        in_specs=[pl.BlockSpec((RB, ci), lambda i: (i, 0)),
                  pl.BlockSpec((8, ci), lambda i: (0, 0)),
                  pl.BlockSpec((1, ci), lambda i: (0, 0)),
                  pl.BlockSpec((1, ci), lambda i: (0, 0)),
                  pl.BlockSpec((RB, ci), lambda i: (i, 0)),
                  pl.BlockSpec((RB, 1), lambda i: (i, 0)),
                  pl.BlockSpec((ci, co), lambda i: (0, 0))],
        out_specs=[pl.BlockSpec((RB, ci), lambda i: (i, 0)),
                   pl.BlockSpec((RB, co), lambda i: (i, 0))],
        out_shape=[jax.ShapeDtypeStruct((N, ci), jnp.float32),
                   jax.ShapeDtypeStruct((N, co), jnp.float32)],
    )(p, st, g, b, skip, ns_col, w2)


def _l2a_body(agg_ref, y2_ref, nd_ref, b_ref, p_ref, st_ref):
    i = pl.program_id(0)
    p = (agg_ref[...] + y2_ref[...]) * nd_ref[...] + b_ref[...]
    p_ref[...] = p

    @pl.when(i == 0)
    def _():
        st_ref[...] = jnp.zeros_like(st_ref)

    st_ref[0, :] += jnp.sum(p, axis=0)
    st_ref[1, :] += jnp.sum(p * p, axis=0)


def _l2a(agg, y2, nd_col, b):
    co = y2.shape[1]
    return pl.pallas_call(
        _l2a_body,
        grid=(NBLK,),
        in_specs=[pl.BlockSpec((RB, co), lambda i: (i, 0)),
                  pl.BlockSpec((RB, co), lambda i: (i, 0)),
                  pl.BlockSpec((RB, 1), lambda i: (i, 0)),
                  pl.BlockSpec((1, co), lambda i: (0, 0))],
        out_specs=[pl.BlockSpec((RB, co), lambda i: (i, 0)),
                   pl.BlockSpec((8, co), lambda i: (0, 0))],
        out_shape=[jax.ShapeDtypeStruct((N, co), jnp.float32),
                   jax.ShapeDtypeStruct((8, co), jnp.float32)],
    )(agg, y2, nd_col, b)


def _l2b_body(p_ref, st_ref, g_ref, bb_ref, x1_ref, skw_ref, skb_ref,
              wg_ref, bg_ref, w3_ref, b3_ref,
              gi_ref, gf_ref, gg_ref, go_ref, sk3_ref):
    sk = jnp.dot(x1_ref[...], skw_ref[...],
                 preferred_element_type=jnp.float32, precision=lax.Precision.HIGHEST) + skb_ref[...]
    x2 = jnp.maximum(_bn(p_ref[...], st_ref, g_ref, bb_ref) + sk, 0.0)
    for k, oref in enumerate((gi_ref, gf_ref, gg_ref, go_ref)):
        oref[...] = jnp.dot(
            x2, wg_ref[:, pl.ds(64 * k, 64)],
            preferred_element_type=jnp.float32,
            precision=lax.Precision.HIGHEST) + bg_ref[:, pl.ds(64 * k, 64)]
    sk3_ref[...] = jnp.dot(x2, w3_ref[...],
                           preferred_element_type=jnp.float32, precision=lax.Precision.HIGHEST) + b3_ref[...]


def _l2b(p, st, g, b, x1, skw, skb, wg, bg, w3, b3):
    return pl.pallas_call(
        _l2b_body,
        grid=(NBLK,),
        in_specs=[pl.BlockSpec((RB, 64), lambda i: (i, 0)),
                  pl.BlockSpec((8, 64), lambda i: (0, 0)),
                  pl.BlockSpec((1, 64), lambda i: (0, 0)),
                  pl.BlockSpec((1, 64), lambda i: (0, 0)),
                  pl.BlockSpec((RB, 128), lambda i: (i, 0)),
                  pl.BlockSpec((128, 64), lambda i: (0, 0)),
                  pl.BlockSpec((1, 64), lambda i: (0, 0)),
                  pl.BlockSpec((64, 256), lambda i: (0, 0)),
                  pl.BlockSpec((1, 256), lambda i: (0, 0)),
                  pl.BlockSpec((64, 64), lambda i: (0, 0)),
                  pl.BlockSpec((1, 64), lambda i: (0, 0))],
        out_specs=[pl.BlockSpec((RB, 64), lambda i: (i, 0)),
                   pl.BlockSpec((RB, 64), lambda i: (i, 0)),
                   pl.BlockSpec((RB, 64), lambda i: (i, 0)),
                   pl.BlockSpec((RB, 64), lambda i: (i, 0)),
                   pl.BlockSpec((RB, 64), lambda i: (i, 0))],
        out_shape=[jax.ShapeDtypeStruct((N, 64), jnp.float32),
                   jax.ShapeDtypeStruct((N, 64), jnp.float32),
                   jax.ShapeDtypeStruct((N, 64), jnp.float32),
                   jax.ShapeDtypeStruct((N, 64), jnp.float32),
                   jax.ShapeDtypeStruct((N, 64), jnp.float32)],
    )(p, st, g, b, x1, skw, skb, wg, bg, w3, b3)


def _lstm_body(gif_ref, gir_ref, gff_ref, gfr_ref, ggf_ref, ggr_ref,
               gof_ref, gor_ref, wb_ref, wr_ref, of_ref, or_ref, hc_ref):
    i = pl.program_id(0)

    @pl.when(i == 0)
    def _():
        hc_ref[...] = jnp.zeros_like(hc_ref)

    f32 = jnp.float32
    bf16 = jnp.bfloat16
    lane = lax.broadcasted_iota(jnp.int32, (1, 64), 1)
    fwd_lane = lane < 32
    wb = [wb_ref[pl.ds(64 * k, 64), :] for k in range(4)]
    wres = [wr_ref[pl.ds(64 * k, 64), :] for k in range(4)]
    h = hc_ref[0:1, :]
    c = hc_ref[1:2, :]

    def step(j, carry):
        h, c = carry
        hb = h.astype(bf16)
        hres = (h - hb.astype(f32)).astype(bf16)
        z = []
        for k, (fr, rr) in enumerate(((gif_ref, gir_ref), (gff_ref, gfr_ref),
                                      (ggf_ref, ggr_ref), (gof_ref, gor_ref))):
            g2 = jnp.where(fwd_lane, fr[pl.ds(j, 1), :],
                           rr[pl.ds(RB - 1 - j, 1), :])
            z.append(g2
                     + jnp.dot(hb, wb[k], preferred_element_type=f32)
                     + jnp.dot(hb, wres[k], preferred_element_type=f32)
                     + jnp.dot(hres, wb[k], preferred_element_type=f32))
        zi, zf, zg, zo = z
        c = jax.nn.sigmoid(zf) * c + jax.nn.sigmoid(zi) * jnp.tanh(zg)
        h = jax.nn.sigmoid(zo) * jnp.tanh(c)
        of_ref[pl.ds(j, 1), :] = h
        or_ref[pl.ds(RB - 1 - j, 1), :] = h
        return (h, c)

    h, c = lax.fori_loop(0, RB, step, (h, c))
    hc_ref[0:1, :] = h
    hc_ref[1:2, :] = c


def _lstm(gi, gf, gg, go, wg):
    wb = wg.astype(jnp.bfloat16)
    wres = (wg - wb.astype(jnp.float32)).astype(jnp.bfloat16)
    gspec_f = pl.BlockSpec((RB, 64), lambda i: (i, 0))
    gspec_r = pl.BlockSpec((RB, 64), lambda i: (NBLK - 1 - i, 0))
    return pl.pallas_call(
        _lstm_body,
        grid=(NBLK,),
        in_specs=[gspec_f, gspec_r, gspec_f, gspec_r,
                  gspec_f, gspec_r, gspec_f, gspec_r,
                  pl.BlockSpec((256, 64), lambda i: (0, 0)),
                  pl.BlockSpec((256, 64), lambda i: (0, 0))],
        out_specs=[pl.BlockSpec((RB, 64), lambda i: (i, 0)),
                   pl.BlockSpec((RB, 64), lambda i: (NBLK - 1 - i, 0))],
        out_shape=[jax.ShapeDtypeStruct((N, 64), jnp.float32),
                   jax.ShapeDtypeStruct((N, 64), jnp.float32)],
        scratch_shapes=[pltpu.VMEM((2, 64), jnp.float32)],
    )(gi, gi, gf, gf, gg, gg, go, go, wb, wres)


def _stat64_body(hf_ref, hr_ref, st_ref):
    i = pl.program_id(0)
    h = jnp.concatenate([hf_ref[:, 0:32], hr_ref[:, 32:64]], axis=1)

    @pl.when(i == 0)
    def _():
        st_ref[...] = jnp.zeros_like(st_ref)

    st_ref[0, :] += jnp.sum(h, axis=0)
    st_ref[1, :] += jnp.sum(h * h, axis=0)


def _stat64(hf, hr):
    return pl.pallas_call(
        _stat64_body,
        grid=(NBLK,),
        in_specs=[pl.BlockSpec((RB, 64), lambda i: (i, 0)),
                  pl.BlockSpec((RB, 64), lambda i: (i, 0))],
        out_specs=pl.BlockSpec((8, 64), lambda i: (0, 0)),
        out_shape=jax.ShapeDtypeStruct((8, 64), jnp.float32),
    )(hf, hr)


def _head_body(hf_ref, hr_ref, st_ref, g_ref, bb_ref, sk3_ref,
               aw1_ref, ab1_ref, aw2_ref, ab2_ref,
               cw1_ref, cb1_ref, cw2_ref, cb2_ref, act_ref, con_ref):
    h = jnp.concatenate([hf_ref[:, 0:32], hr_ref[:, 32:64]], axis=1)
    x3 = jnp.maximum(_bn(h, st_ref, g_ref, bb_ref) + sk3_ref[...], 0.0)
    a = jnp.maximum(jnp.dot(x3, aw1_ref[...],
                            preferred_element_type=jnp.float32, precision=lax.Precision.HIGHEST)
                    + ab1_ref[...], 0.0)
    act_ref[...] = jnp.dot(a, aw2_ref[...],
                           preferred_element_type=jnp.float32, precision=lax.Precision.HIGHEST) + ab2_ref[...]
    c = jnp.maximum(jnp.dot(x3, cw1_ref[...],
                            preferred_element_type=jnp.float32, precision=lax.Precision.HIGHEST)
                    + cb1_ref[...], 0.0)
    con_ref[...] = jnp.dot(c, cw2_ref[...],
                           preferred_element_type=jnp.float32, precision=lax.Precision.HIGHEST) + cb2_ref[...]


def _heads(hf, hr, st, g, b, sk3, aw1, ab1, aw2, ab2, cw1, cb1, cw2, cb2):
    return pl.pallas_call(
        _head_body,
        grid=(NBLK,),
        in_specs=[pl.BlockSpec((RB, 64), lambda i: (i, 0)),
                  pl.BlockSpec((RB, 64), lambda i: (i, 0)),
                  pl.BlockSpec((8, 64), lambda i: (0, 0)),
                  pl.BlockSpec((1, 64), lambda i: (0, 0)),
                  pl.BlockSpec((1, 64), lambda i: (0, 0)),
                  pl.BlockSpec((RB, 64), lambda i: (i, 0)),
                  pl.BlockSpec((64, 64), lambda i: (0, 0)),
                  pl.BlockSpec((1, 64), lambda i: (0, 0)),
                  pl.BlockSpec((64, 1), lambda i: (0, 0)),
                  pl.BlockSpec((1, 1), lambda i: (0, 0)),
                  pl.BlockSpec((64, 64), lambda i: (0, 0)),
                  pl.BlockSpec((1, 64), lambda i: (0, 0)),
                  pl.BlockSpec((64, 1), lambda i: (0, 0)),
                  pl.BlockSpec((1, 1), lambda i: (0, 0))],
        out_specs=[pl.BlockSpec((RB, 1), lambda i: (i, 0)),
                   pl.BlockSpec((RB, 1), lambda i: (i, 0))],
        out_shape=[jax.ShapeDtypeStruct((N, 1), jnp.float32),
                   jax.ShapeDtypeStruct((N, 1), jnp.float32)],
    )(hf, hr, st, g, b, sk3, aw1, ab1, aw2, ab2, cw1, cb1, cw2, cb2)


def kernel(feat, node_id, edge_index, emb_table, W1, b1, W2, b2, sc1_w, sc1_b,
           sc2_w, sc2_b, sc3_w, sc3_b, bn1_g, bn1_b, bn2_g, bn2_b, bn3_g,
           bn3_b, Wih_f, Whh_f, bih_f, bhh_f, Wih_r, Whh_r, bih_r, bhh_r,
           act1_w, act1_b, act2_w, act2_b, con1_w, con1_b, con2_w, con2_b):
    f32 = jnp.float32
    src = edge_index[0].astype(jnp.int32)
    dst = edge_index[1].astype(jnp.int32)
    pad = jnp.full((E_PAD - E,), TRASH_NODE, jnp.int32)
    src2d = jnp.concatenate([src, pad]).reshape(NCHUNKS, ECHUNK)
    dst2d = jnp.concatenate([dst, pad]).reshape(NCHUNKS, ECHUNK)
    nid2d = jnp.pad(node_id.astype(jnp.int32),
                    (0, NID_PAD - N)).reshape(NID_CHUNKS, ECHUNK)
    emb_pad = jnp.pad(emb_table.astype(f32), ((0, 0), (0, 6)))

    emb_rows = _emb_call(nid2d, emb_pad)

    ones_pad = jnp.pad(jnp.ones((NROWS_PAD, 1), f32), ((0, 0), (0, 15)))
    zeros16 = jnp.zeros((CPY, 16), f32)
    degi_2 = _agg_call(16, ones_pad, src2d, dst2d, zeros16)
    dego_2 = _agg_call(16, ones_pad, dst2d, src2d, zeros16)
    degi_col = jnp.concatenate([degi_2[0, :HALF, :1], degi_2[1, :HALF, :1]])
    dego_col = jnp.concatenate([dego_2[0, :HALF, :1], dego_2[1, :HALF, :1]])
    ns_col, nd_col = _norms(dego_col, degi_col)

    emb10 = emb_rows[:N, :10]
    xs = _make_xs(feat, emb10, ns_col)
    xs_pad = jnp.pad(xs, ((0, NROWS_PAD - N), (0, 0)))

    sc1_wa = sc1_w[:35]
    sc1_wb = sc1_w[35:45]
    skip1 = _skip1(feat, emb10, sc1_wa, sc1_wb, sc1_b.reshape(1, -1))

    zeros48 = jnp.zeros((CPY, 48), f32)
    agg1_2 = _agg_call(48, xs_pad, src2d, dst2d, zeros48)
    agg1 = jnp.concatenate([agg1_2[0, :HALF], agg1_2[1, :HALF]], axis=0)

    W1p = jnp.pad(W1, ((0, 3), (0, 0)))
    p1, st1 = _l1a(agg1, xs, nd_col, W1p, b1.reshape(1, -1))
    x1, y2 = _l1b(p1, st1, bn1_g.reshape(1, -1), bn1_b.reshape(1, -1),
                  skip1, ns_col, W2)

    y2_pad = jnp.pad(y2, ((0, NROWS_PAD - N), (0, 0)))
    zeros64 = jnp.zeros((CPY, 64), f32)
    agg2_2 = _agg_call(64, y2_pad, src2d, dst2d, zeros64)
    agg2 = jnp.concatenate([agg2_2[0, :HALF], agg2_2[1, :HALF]], axis=0)

    p2, st2 = _l2a(agg2, y2, nd_col, b2.reshape(1, -1))

    bf_all = (bih_f + bhh_f).reshape(1, -1)
    br_all = (bih_r + bhh_r).reshape(1, -1)
    wg_in = jnp.concatenate(
        [jnp.concatenate([Wih_f[:, 32 * k:32 * k + 32],
                          Wih_r[:, 32 * k:32 * k + 32]], axis=1)
         for k in range(4)], axis=1)
    bg_in = jnp.concatenate(
        [jnp.concatenate([bf_all[:, 32 * k:32 * k + 32],
                          br_all[:, 32 * k:32 * k + 32]], axis=1)
         for k in range(4)], axis=1)
    gi, gfg, gg, go, sk3 = _l2b(
        p2, st2, bn2_g.reshape(1, -1), bn2_b.reshape(1, -1), x1,
        sc2_w, sc2_b.reshape(1, -1), wg_in, bg_in,
        sc3_w, sc3_b.reshape(1, -1))

    z32 = jnp.zeros((32, 32), f32)
    wg_rec = jnp.concatenate(
        [jnp.concatenate(
            [jnp.concatenate([Whh_f[:, 32 * k:32 * k + 32], z32], axis=1),
             jnp.concatenate([z32, Whh_r[:, 32 * k:32 * k + 32]], axis=1)],
            axis=0)
         for k in range(4)], axis=0)
    ohf, ohr = _lstm(gi, gfg, gg, go, wg_rec)
    hs_f, hs_r = ohf, ohr
    st3 = _stat64(hs_f, hs_r)
    active, consume = _heads(
        hs_f, hs_r, st3, bn3_g.reshape(1, -1), bn3_b.reshape(1, -1), sk3,
        act1_w, act1_b.reshape(1, -1), act2_w, act2_b.reshape(1, -1),
        con1_w, con1_b.reshape(1, -1), con2_w, con2_b.reshape(1, -1))
    return (active, consume)
